# baseline (speedup 1.0000x reference)
import functools

import jax
import jax.numpy as jnp
from jax import lax
from jax.experimental import pallas as pl
from jax.experimental.pallas import tpu as pltpu
from jax.experimental.pallas import tpu_sc as plsc

N = 10000
E = 320000
D = 128
DH = D // 2

NC = 2
NS = 16
NW = NC * NS
EPS = E // NS
EPH = EPS // NC
CH = 128
NFULL = EPS // CH
TAIL = EPS - NFULL * CH
LANES = 16
N_PAD = 10240
RPT = N_PAD // NS
RSTG = RPT // 8


def _sc_aggregate(src, dst, feat_a, feat_b, zeros_agg, zeros_hist):
    mesh = plsc.VectorSubcoreMesh(core_axis_name="c", subcore_axis_name="s")

    @functools.partial(
        pl.kernel,
        mesh=mesh,
        compiler_params=pltpu.CompilerParams(use_tc_tiling_on_sc=False,
                                             needs_layout_passes=False),
        out_type=[
            jax.ShapeDtypeStruct((NC, N_PAD, DH), jnp.float32),
            jax.ShapeDtypeStruct((NW, N_PAD), jnp.float32),
        ],
        scratch_types=[
            pltpu.VMEM((EPS,), jnp.int32),
            pltpu.VMEM((EPS,), jnp.int32),
            pltpu.VMEM((CH, DH), jnp.float32),
            pltpu.VMEM((CH, DH), jnp.float32),
            pltpu.VMEM((CH, DH), jnp.float32),
            pltpu.VMEM((CH, DH), jnp.float32),
            pltpu.VMEM((N_PAD,), jnp.float32),
            pltpu.VMEM((RSTG, DH), jnp.float32),
            pltpu.VMEM_SHARED((N_PAD, DH), jnp.float32),
            pltpu.SemaphoreType.DMA,
            pltpu.SemaphoreType.DMA,
            pltpu.SemaphoreType.DMA,
            pltpu.SemaphoreType.DMA,
            pltpu.SemaphoreType.DMA,
            pltpu.SemaphoreType.DMA,
            pltpu.SemaphoreType.DMA,
            pltpu.SemaphoreType.DMA,
        ],
    )
    def agg_kernel(src_hbm, dst_hbm, fa_hbm, fb_hbm, zagg_hbm, zhist_hbm,
                   agg_out, cnt_out,
                   sidx, didx, rows0, rows1, rows2, rows3, hist, zbuf,
                   agg_sh, gs0, gs1, gs2, gs3, ss0, ss1, ss2, ss3):
        cid = lax.axis_index("c")
        sid = lax.axis_index("s")
        wid = sid * NC + cid
        row0 = sid * RPT
        base = sid * EPS

        pltpu.sync_copy(src_hbm.at[pl.ds(base, EPS)], sidx)
        pltpu.sync_copy(dst_hbm.at[pl.ds(base, EPS)], didx)

        bufs = (rows0, rows1, rows2, rows3)
        gsems = (gs0, gs1, gs2, gs3)
        ssems = (ss0, ss1, ss2, ss3)

        def sl(i):
            return pl.ds(i * CH, CH)

        def one_pass(fsrc, hbase):
            pltpu.sync_copy(zhist_hbm, hist)
            ones_vec = jnp.ones((LANES,), jnp.float32)

            @pl.loop(0, EPH // LANES)
            def _(k):
                iv = didx[pl.ds(hbase + k * LANES, LANES)]
                plsc.addupdate_scatter(hist, [iv], ones_vec)

            pltpu.sync_copy(hist, cnt_out.at[wid])

            pltpu.sync_copy(zagg_hbm, zbuf)
            for k in range(RPT // RSTG):
                pltpu.sync_copy(zbuf,
                                agg_sh.at[pl.ds(row0 + k * RSTG, RSTG)])
            plsc.subcore_barrier()

            def g_start(i, b):
                pltpu.async_copy(fsrc.at[sidx.at[sl(i)]], bufs[b], gsems[b])

            def g_wait(i, b):
                pltpu.make_async_copy(fsrc.at[sidx.at[sl(i)]], bufs[b],
                                      gsems[b]).wait()

            def s_start(i, b):
                pltpu.async_copy(bufs[b], agg_sh.at[didx.at[sl(i)]],
                                 ssems[b], add=True)

            def s_wait(i, b):
                pltpu.make_async_copy(bufs[b], agg_sh.at[didx.at[sl(i)]],
                                      ssems[b]).wait()

            def body(j, b, do_swait, do_gstart):
                g_wait(j, b)
                s_start(j, b)
                if do_swait:
                    s_wait(j - 1, (b - 1) % 4)
                if do_gstart:
                    g_start(j + 3, (b + 3) % 4)

            for i in range(3):
                g_start(i, i)
            for j in range(4):
                body(j, j, j >= 1, True)

            @pl.loop(1, NFULL // 4 - 1)
            def _(g):
                for p in range(4):
                    body(4 * g + p, p, True, True)

            for j in range(4 * (NFULL // 4 - 1), NFULL):
                body(j, j % 4, True, j + 3 < NFULL)
            s_wait(NFULL - 1, (NFULL - 1) % 4)

            if TAIL:
                toff = pl.ds(NFULL * CH, TAIL)
                tbuf = rows0.at[pl.ds(0, TAIL)]
                pltpu.sync_copy(fsrc.at[sidx.at[toff]], tbuf)
                pltpu.sync_copy(tbuf, agg_sh.at[didx.at[toff]], add=True)

            plsc.subcore_barrier()
            for k in range(RPT // RSTG):
                r0 = row0 + k * RSTG
                pltpu.sync_copy(agg_sh.at[pl.ds(r0, RSTG)], zbuf)
                pltpu.sync_copy(zbuf, agg_out.at[cid, pl.ds(r0, RSTG)])
            plsc.subcore_barrier()

        @pl.when(cid == 0)
        def _():
            one_pass(fa_hbm, 0)

        @pl.when(cid == 1)
        def _():
            one_pass(fb_hbm, EPH)

    return agg_kernel(src, dst, feat_a, feat_b, zeros_agg, zeros_hist)


def _self_body(feat_ref, wr_ref, bl_ref, out_ref):
    out_ref[...] = (
        lax.dot_general(feat_ref[...], wr_ref[...], (((1,), (1,)), ((), ())),
                        preferred_element_type=jnp.float32)
        + bl_ref[...]
    )


def _tc_self(feat_pad, W_r, b_l2d):
    BN = 1024
    return pl.pallas_call(
        _self_body,
        grid=(N_PAD // BN,),
        in_specs=[
            pl.BlockSpec((BN, D), lambda i: (i, 0)),
            pl.BlockSpec((D, D), lambda i: (0, 0)),
            pl.BlockSpec((1, D), lambda i: (0, 0)),
        ],
        out_specs=pl.BlockSpec((BN, D), lambda i: (i, 0)),
        out_shape=jax.ShapeDtypeStruct((N_PAD, D), jnp.float32),
    )(feat_pad, W_r, b_l2d)


def _combine_body(agg_ref, cnt_ref, self_ref, wl_ref, out_ref):
    cnt = jnp.sum(cnt_ref[...], axis=0)
    inv = (1.0 / jnp.maximum(cnt, 1.0))[:, None]
    mean_a = agg_ref[0] * inv
    mean_b = agg_ref[1] * inv
    wl = wl_ref[...]
    out_ref[...] = (
        lax.dot_general(mean_a, wl[:, :DH], (((1,), (1,)), ((), ())),
                        preferred_element_type=jnp.float32)
        + lax.dot_general(mean_b, wl[:, DH:], (((1,), (1,)), ((), ())),
                          preferred_element_type=jnp.float32)
        + self_ref[...]
    )


def _tc_combine(agg, cnt_parts, self_term, W_l):
    BN = 1024
    return pl.pallas_call(
        _combine_body,
        grid=(N_PAD // BN,),
        in_specs=[
            pl.BlockSpec((NC, BN, DH), lambda i: (0, i, 0)),
            pl.BlockSpec((NW, BN), lambda i: (0, i)),
            pl.BlockSpec((BN, D), lambda i: (i, 0)),
            pl.BlockSpec((D, D), lambda i: (0, 0)),
        ],
        out_specs=pl.BlockSpec((BN, D), lambda i: (i, 0)),
        out_shape=jax.ShapeDtypeStruct((N_PAD, D), jnp.float32),
    )(agg, cnt_parts, self_term, W_l)


def kernel(feature, edge_index, W_l, b_l, W_r):
    src = edge_index[0].astype(jnp.int32)
    dst = edge_index[1].astype(jnp.int32)
    feat_pad = jnp.pad(feature, ((0, N_PAD - N), (0, 0)))
    feat_a = feature[:, :DH]
    feat_b = feature[:, DH:]
    zeros_agg = jnp.zeros((RSTG, DH), jnp.float32)
    zeros_hist = jnp.zeros((N_PAD,), jnp.float32)
    self_term = _tc_self(feat_pad, W_r, b_l.reshape(1, D))
    agg, cnt_parts = _sc_aggregate(src, dst, feat_a, feat_b,
                                   zeros_agg, zeros_hist)
    out_pad = _tc_combine(agg, cnt_parts, self_term, W_l)
    return out_pad[:N]


# speedup vs baseline: 1.0089x; 1.0089x over previous
import functools

import jax
import jax.numpy as jnp
from jax import lax
from jax.experimental import pallas as pl
from jax.experimental.pallas import tpu as pltpu
from jax.experimental.pallas import tpu_sc as plsc

N = 10000
E = 320000
D = 128
DH = D // 2

NC = 2
NS = 16
NW = NC * NS
EPS = E // NS
EPH = EPS // NC
CH = 128
NFULL = EPS // CH
TAIL = EPS - NFULL * CH
LANES = 16
N_PAD = 10240
RPT = N_PAD // NS
RSTG = RPT // 8


def _sc_aggregate(src, dst, feat_a, feat_b, zeros_agg, zeros_hist):
    mesh = plsc.VectorSubcoreMesh(core_axis_name="c", subcore_axis_name="s")

    @functools.partial(
        pl.kernel,
        mesh=mesh,
        compiler_params=pltpu.CompilerParams(use_tc_tiling_on_sc=False,
                                             needs_layout_passes=False),
        out_type=[
            jax.ShapeDtypeStruct((NC, N_PAD, DH), jnp.float32),
            jax.ShapeDtypeStruct((NW, N_PAD), jnp.float32),
        ],
        scratch_types=[
            pltpu.VMEM((EPS,), jnp.int32),
            pltpu.VMEM((EPS,), jnp.int32),
            pltpu.VMEM((CH, DH), jnp.float32),
            pltpu.VMEM((CH, DH), jnp.float32),
            pltpu.VMEM((CH, DH), jnp.float32),
            pltpu.VMEM((CH, DH), jnp.float32),
            pltpu.VMEM((N_PAD,), jnp.float32),
            pltpu.VMEM((RSTG, DH), jnp.float32),
            pltpu.VMEM_SHARED((N_PAD, DH), jnp.float32),
            pltpu.SemaphoreType.DMA,
            pltpu.SemaphoreType.DMA,
            pltpu.SemaphoreType.DMA,
            pltpu.SemaphoreType.DMA,
            pltpu.SemaphoreType.DMA,
            pltpu.SemaphoreType.DMA,
            pltpu.SemaphoreType.DMA,
            pltpu.SemaphoreType.DMA,
        ],
    )
    def agg_kernel(src_hbm, dst_hbm, fa_hbm, fb_hbm, zagg_hbm, zhist_hbm,
                   agg_out, cnt_out,
                   sidx, didx, rows0, rows1, rows2, rows3, hist, zbuf,
                   agg_sh, gs0, gs1, gs2, gs3, ss0, ss1, ss2, ss3):
        cid = lax.axis_index("c")
        sid = lax.axis_index("s")
        wid = sid * NC + cid
        row0 = sid * RPT
        base = sid * EPS

        pltpu.sync_copy(src_hbm.at[pl.ds(base, EPS)], sidx)
        pltpu.sync_copy(dst_hbm.at[pl.ds(base, EPS)], didx)

        bufs = (rows0, rows1, rows2, rows3)
        gsems = (gs0, gs1, gs2, gs3)
        ssems = (ss0, ss1, ss2, ss3)

        def sl(i):
            return pl.ds(i * CH, CH)

        def one_pass(fsrc, hbase):
            pltpu.sync_copy(zhist_hbm, hist)
            ones_vec = jnp.ones((LANES,), jnp.float32)

            @pl.loop(0, EPH // LANES)
            def _(k):
                iv = didx[pl.ds(hbase + k * LANES, LANES)]
                plsc.addupdate_scatter(hist, [iv], ones_vec)

            pltpu.sync_copy(hist, cnt_out.at[wid])

            pltpu.sync_copy(zagg_hbm, zbuf)
            for k in range(RPT // RSTG):
                pltpu.sync_copy(zbuf,
                                agg_sh.at[pl.ds(row0 + k * RSTG, RSTG)])
            plsc.subcore_barrier()

            def g_start(i, b):
                pltpu.async_copy(fsrc.at[sidx.at[sl(i)]], bufs[b], gsems[b])

            def g_wait(i, b):
                pltpu.make_async_copy(fsrc.at[sidx.at[sl(i)]], bufs[b],
                                      gsems[b]).wait()

            def s_start(i, b):
                pltpu.async_copy(bufs[b], agg_sh.at[didx.at[sl(i)]],
                                 ssems[b], add=True)

            def s_wait(i, b):
                pltpu.make_async_copy(bufs[b], agg_sh.at[didx.at[sl(i)]],
                                      ssems[b]).wait()

            def body(j, b, do_swait, do_gstart):
                g_wait(j, b)
                s_start(j, b)
                if do_swait:
                    s_wait(j - 1, (b - 1) % 4)
                if do_gstart:
                    g_start(j + 3, (b + 3) % 4)

            for i in range(3):
                g_start(i, i)
            for j in range(4):
                body(j, j, j >= 1, True)

            @pl.loop(1, NFULL // 4 - 1)
            def _(g):
                for p in range(4):
                    body(4 * g + p, p, True, True)

            for j in range(4 * (NFULL // 4 - 1), NFULL):
                body(j, j % 4, True, j + 3 < NFULL)
            s_wait(NFULL - 1, (NFULL - 1) % 4)

            if TAIL:
                toff = pl.ds(NFULL * CH, TAIL)
                tbuf = rows0.at[pl.ds(0, TAIL)]
                pltpu.sync_copy(fsrc.at[sidx.at[toff]], tbuf)
                pltpu.sync_copy(tbuf, agg_sh.at[didx.at[toff]], add=True)

            plsc.subcore_barrier()
            for k in range(RPT // RSTG):
                r0 = row0 + k * RSTG
                pltpu.sync_copy(agg_sh.at[pl.ds(r0, RSTG)], zbuf)
                pltpu.sync_copy(zbuf, agg_out.at[cid, pl.ds(r0, RSTG)])
            plsc.subcore_barrier()

        @pl.when(cid == 0)
        def _():
            one_pass(fa_hbm, 0)

        @pl.when(cid == 1)
        def _():
            one_pass(fb_hbm, EPH)

    return agg_kernel(src, dst, feat_a, feat_b, zeros_agg, zeros_hist)


def _combine_body(agg_ref, cnt_ref, feat_ref, wl_ref, bl_ref, wr_ref,
                  out_ref):
    cnt = jnp.sum(cnt_ref[...], axis=0)
    inv = (1.0 / jnp.maximum(cnt, 1.0))[:, None]
    mean_a = agg_ref[0] * inv
    mean_b = agg_ref[1] * inv
    wl = wl_ref[...]
    out_ref[...] = (
        lax.dot_general(mean_a, wl[:, :DH], (((1,), (1,)), ((), ())),
                        preferred_element_type=jnp.float32)
        + lax.dot_general(mean_b, wl[:, DH:], (((1,), (1,)), ((), ())),
                          preferred_element_type=jnp.float32)
        + lax.dot_general(feat_ref[...], wr_ref[...], (((1,), (1,)), ((), ())),
                          preferred_element_type=jnp.float32)
        + bl_ref[...]
    )


def _tc_combine(agg, cnt_parts, feat_pad, W_l, b_l2d, W_r):
    BN = 1024
    return pl.pallas_call(
        _combine_body,
        grid=(N_PAD // BN,),
        in_specs=[
            pl.BlockSpec((NC, BN, DH), lambda i: (0, i, 0)),
            pl.BlockSpec((NW, BN), lambda i: (0, i)),
            pl.BlockSpec((BN, D), lambda i: (i, 0)),
            pl.BlockSpec((D, D), lambda i: (0, 0)),
            pl.BlockSpec((1, D), lambda i: (0, 0)),
            pl.BlockSpec((D, D), lambda i: (0, 0)),
        ],
        out_specs=pl.BlockSpec((BN, D), lambda i: (i, 0)),
        out_shape=jax.ShapeDtypeStruct((N_PAD, D), jnp.float32),
    )(agg, cnt_parts, feat_pad, W_l, b_l2d, W_r)


def kernel(feature, edge_index, W_l, b_l, W_r):
    src = edge_index[0].astype(jnp.int32)
    dst = edge_index[1].astype(jnp.int32)
    feat_pad = jnp.pad(feature, ((0, N_PAD - N), (0, 0)))
    feat_a = feature[:, :DH]
    feat_b = feature[:, DH:]
    zeros_agg = jnp.zeros((RSTG, DH), jnp.float32)
    zeros_hist = jnp.zeros((N_PAD,), jnp.float32)
    agg, cnt_parts = _sc_aggregate(src, dst, feat_a, feat_b,
                                   zeros_agg, zeros_hist)
    out_pad = _tc_combine(agg, cnt_parts, feat_pad, W_l,
                          b_l.reshape(1, D), W_r)
    return out_pad[:N]
